# baseline (speedup 1.0000x reference)
import functools

import jax
import jax.numpy as jnp
from jax import lax
from jax.experimental import pallas as pl
from jax.experimental.pallas import tpu as pltpu
from jax.experimental.pallas import tpu_sc as plsc

VOCAB = 1000000
EMBED_DIM = 64
BATCH = 4096
SEQ = 200

_N = BATCH * SEQ
_NC = 2
_NS = 16
_NW = _NC * _NS
_PER_W = _N // _NW
_IDX_ROW = 128
_K = 8
_CHUNK = _K * _IDX_ROW
_STEPS = _PER_W // _CHUNK


@functools.partial(
    pl.kernel,
    mesh=plsc.VectorSubcoreMesh(core_axis_name="c", subcore_axis_name="s"),
    compiler_params=pltpu.CompilerParams(use_tc_tiling_on_sc=False),
    out_type=jax.ShapeDtypeStruct((_N, EMBED_DIM), jnp.float32),
    scratch_types=[
        pltpu.VMEM((_K, _IDX_ROW), jnp.int32),
        pltpu.VMEM((_CHUNK, EMBED_DIM), jnp.float32),
        pltpu.SemaphoreType.DMA,
    ],
)
def _gather_kernel(table_hbm, idx_hbm, out_hbm, idx_v, rows_v, sem):
    wid = lax.axis_index("s") * _NC + lax.axis_index("c")
    row_base = wid * (_PER_W // _IDX_ROW)
    base = wid * _PER_W

    def step(i, carry):
        pltpu.sync_copy(idx_hbm.at[pl.ds(row_base + i * _K, _K)], idx_v)
        copies = [
            pltpu.async_copy(
                table_hbm.at[idx_v.at[j]],
                rows_v.at[pl.ds(j * _IDX_ROW, _IDX_ROW)],
                sem,
            )
            for j in range(_K)
        ]
        for c in copies:
            c.wait()
        pltpu.sync_copy(rows_v, out_hbm.at[pl.ds(base + i * _CHUNK, _CHUNK)])
        return carry

    lax.fori_loop(0, _STEPS, step, 0)


def kernel(sentence, word_embed):
    idx = sentence.reshape(_N // _IDX_ROW, _IDX_ROW).astype(jnp.int32)
    out = _gather_kernel(word_embed, idx)
    return out.reshape(BATCH, SEQ, EMBED_DIM)


# speedup vs baseline: 1.0107x; 1.0107x over previous
import functools

import jax
import jax.numpy as jnp
from jax import lax
from jax.experimental import pallas as pl
from jax.experimental.pallas import tpu as pltpu
from jax.experimental.pallas import tpu_sc as plsc

VOCAB = 1000000
EMBED_DIM = 64
BATCH = 4096
SEQ = 200

_N = BATCH * SEQ
_NC = 2
_NS = 16
_NW = _NC * _NS
_PER_W = _N // _NW
_IDX_ROW = 128
_K = 5
_CHUNK = _K * _IDX_ROW
_STEPS = _PER_W // _CHUNK


@functools.partial(
    pl.kernel,
    mesh=plsc.VectorSubcoreMesh(core_axis_name="c", subcore_axis_name="s"),
    compiler_params=pltpu.CompilerParams(use_tc_tiling_on_sc=False),
    out_type=jax.ShapeDtypeStruct((_N, EMBED_DIM), jnp.float32),
    scratch_types=[
        pltpu.VMEM((2, _K, _IDX_ROW), jnp.int32),
        pltpu.VMEM((2, _CHUNK, EMBED_DIM), jnp.float32),
        pltpu.SemaphoreType.DMA,
        pltpu.SemaphoreType.DMA,
    ],
)
def _gather_kernel(table_hbm, idx_hbm, out_hbm, idx_v, rows_v, gsem, ssem):
    wid = lax.axis_index("s") * _NC + lax.axis_index("c")
    row_base = wid * (_PER_W // _IDX_ROW)
    base = wid * _PER_W

    def fire_gathers(c, b):
        pltpu.sync_copy(idx_hbm.at[pl.ds(row_base + c * _K, _K)], idx_v.at[b])
        for j in range(_K):
            pltpu.async_copy(
                table_hbm.at[idx_v.at[b].at[j]],
                rows_v.at[b].at[pl.ds(j * _IDX_ROW, _IDX_ROW)],
                gsem,
            )

    def wait_gathers(b):
        for j in range(_K):
            pltpu.make_async_copy(
                table_hbm.at[idx_v.at[b].at[j]],
                rows_v.at[b].at[pl.ds(j * _IDX_ROW, _IDX_ROW)],
                gsem,
            ).wait()

    def fire_store(c, b):
        pltpu.async_copy(
            rows_v.at[b], out_hbm.at[pl.ds(base + c * _CHUNK, _CHUNK)], ssem
        )

    def wait_store(c, b):
        pltpu.make_async_copy(
            rows_v.at[b], out_hbm.at[pl.ds(base + c * _CHUNK, _CHUNK)], ssem
        ).wait()

    fire_gathers(0, 0)

    def step(c, carry):
        b = lax.rem(c, 2)
        nb = 1 - b

        @pl.when(c + 1 < _STEPS)
        def _prefetch():
            @pl.when(c >= 1)
            def _():
                wait_store(c - 1, nb)
            fire_gathers(c + 1, nb)

        wait_gathers(b)
        fire_store(c, b)
        return carry

    lax.fori_loop(0, _STEPS, step, 0)
    wait_store(_STEPS - 2, 0)
    wait_store(_STEPS - 1, 1)


def kernel(sentence, word_embed):
    idx = sentence.reshape(_N // _IDX_ROW, _IDX_ROW).astype(jnp.int32)
    out = _gather_kernel(word_embed, idx)
    return out.reshape(BATCH, SEQ, EMBED_DIM)


# speedup vs baseline: 1.3400x; 1.3258x over previous
import functools

import jax
import jax.numpy as jnp
from jax import lax
from jax.experimental import pallas as pl
from jax.experimental.pallas import tpu as pltpu
from jax.experimental.pallas import tpu_sc as plsc

VOCAB = 1000000
EMBED_DIM = 64
BATCH = 4096
SEQ = 200

_N = BATCH * SEQ
_NC = 2
_NS = 16
_NW = _NC * _NS
_PER_W = _N // _NW
_IDX_ROW = 128
_K = 5
_CHUNK = _K * _IDX_ROW
_STEPS = _PER_W // _CHUNK
_PAD = 2 * EMBED_DIM

assert _PER_W % _CHUNK == 0 and _STEPS % 2 == 0


@functools.partial(
    pl.kernel,
    mesh=plsc.VectorSubcoreMesh(core_axis_name="c", subcore_axis_name="s"),
    compiler_params=pltpu.CompilerParams(use_tc_tiling_on_sc=False),
    out_type=jax.ShapeDtypeStruct((_N, _PAD), jnp.float32),
    scratch_types=[
        pltpu.VMEM((2, _K, _IDX_ROW), jnp.int32),
        pltpu.VMEM((2, _CHUNK, EMBED_DIM), jnp.float32),
        pltpu.SemaphoreType.DMA,
        pltpu.SemaphoreType.DMA,
    ],
)
def _gather_kernel(table_hbm, idx_hbm, out_hbm, idx_v, rows_v, gsem, ssem):
    wid = lax.axis_index("s") * _NC + lax.axis_index("c")
    row_base = wid * (_PER_W // _IDX_ROW)
    base = wid * _PER_W

    def fire_gathers(c, b):
        pltpu.sync_copy(idx_hbm.at[pl.ds(row_base + c * _K, _K)], idx_v.at[b])
        for j in range(_K):
            pltpu.async_copy(
                table_hbm.at[idx_v.at[b].at[j]],
                rows_v.at[b].at[pl.ds(j * _IDX_ROW, _IDX_ROW)],
                gsem,
            )

    def wait_gathers(b):
        for j in range(_K):
            pltpu.make_async_copy(
                table_hbm.at[idx_v.at[b].at[j]],
                rows_v.at[b].at[pl.ds(j * _IDX_ROW, _IDX_ROW)],
                gsem,
            ).wait()

    def fire_store(c, b):
        pltpu.async_copy(
            rows_v.at[b],
            out_hbm.at[pl.ds(base + c * _CHUNK, _CHUNK), pl.ds(0, EMBED_DIM)],
            ssem,
        )

    def wait_store(c, b):
        pltpu.make_async_copy(
            rows_v.at[b],
            out_hbm.at[pl.ds(base + c * _CHUNK, _CHUNK), pl.ds(0, EMBED_DIM)],
            ssem,
        ).wait()

    fire_gathers(0, 0)

    def step(c, carry):
        b = lax.rem(c, 2)
        nb = 1 - b

        @pl.when(c + 1 < _STEPS)
        def _prefetch():
            @pl.when(c >= 1)
            def _():
                wait_store(c - 1, nb)
            fire_gathers(c + 1, nb)

        wait_gathers(b)
        fire_store(c, b)
        return carry

    lax.fori_loop(0, _STEPS, step, 0)
    wait_store(_STEPS - 2, 0)
    wait_store(_STEPS - 1, 1)


def kernel(sentence, word_embed):
    idx = sentence.reshape(_N // _IDX_ROW, _IDX_ROW).astype(jnp.int32)
    out = _gather_kernel(word_embed, idx)
    return out[:, :EMBED_DIM].reshape(BATCH, SEQ, EMBED_DIM)
